# Initial kernel scaffold; baseline (speedup 1.0000x reference)
import jax
import jax.numpy as jnp
from jax.experimental import pallas as pl


def kernel(x, table):
    raise NotImplementedError("write your pallas kernel here")


# speedup vs baseline: 1.4450x; 1.4450x over previous
import jax
import jax.numpy as jnp
from jax import lax
from jax.experimental import pallas as pl
from jax.experimental.pallas import tpu as pltpu
from jax.experimental.pallas import tpu_sc as plsc

NUM_WORDS = 100000
EMBED_DIM = 300

NC = 2
NS = 16
NW = NC * NS

TOT = 4096 * 50
PER_W = TOT // NW
CH = 128
NCH = PER_W // CH
TAIL = EMBED_DIM - 256


def _gather_body(table_hbm, tail_hbm, idx_hbm, out_hbm, idx_v, rows_v, b2, sem):
    wid = lax.axis_index("s") * NC + lax.axis_index("c")
    base = wid * PER_W
    lanes = lax.iota(jnp.int32, 16)
    tail_mask = lanes < (TAIL - 32)

    def chunk(i, carry):
        off = pl.multiple_of(base + i * CH, CH)
        pltpu.sync_copy(idx_hbm.at[pl.ds(off, CH)], idx_v)
        c0 = pltpu.async_copy(
            table_hbm.at[idx_v, pl.ds(0, 128)], rows_v.at[:, pl.ds(0, 128)], sem)
        c1 = pltpu.async_copy(
            table_hbm.at[idx_v, pl.ds(128, 128)], rows_v.at[:, pl.ds(128, 128)],
            sem)
        c2 = pltpu.async_copy(tail_hbm.at[idx_v], b2, sem)
        c0.wait()
        c1.wait()
        c2.wait()

        def row_fix(j, c):
            jv = jnp.full((16,), j, jnp.int32)
            v0 = b2[j, pl.ds(0, 16)]
            v1 = b2[j, pl.ds(16, 16)]
            v2 = b2[j, pl.ds(32, 16)]
            plsc.store_scatter(rows_v, [jv, lanes + 256], v0)
            plsc.store_scatter(rows_v, [jv, lanes + 272], v1)
            plsc.store_scatter(rows_v, [jv, lanes + 288], v2, mask=tail_mask)
            return c

        lax.fori_loop(0, CH, row_fix, 0)
        pltpu.sync_copy(rows_v, out_hbm.at[pl.ds(off, CH)])
        return carry

    lax.fori_loop(0, NCH, chunk, 0)


_gather = pl.kernel(
    _gather_body,
    out_type=jax.ShapeDtypeStruct((TOT, EMBED_DIM), jnp.float32),
    mesh=plsc.VectorSubcoreMesh(
        core_axis_name="c", subcore_axis_name="s", num_cores=NC, num_subcores=NS
    ),
    scratch_types=[
        pltpu.VMEM((CH,), jnp.int32),
        pltpu.VMEM((CH, EMBED_DIM), jnp.float32),
        pltpu.VMEM((CH, 128), jnp.float32),
        pltpu.SemaphoreType.DMA,
    ],
    compiler_params=pltpu.CompilerParams(needs_layout_passes=False),
)


@jax.jit
def kernel(x, table):
    idx = x.reshape(-1).astype(jnp.int32)
    tail = jnp.pad(lax.slice(table, (0, 256), (NUM_WORDS, EMBED_DIM)),
                   ((0, 0), (0, 128 - TAIL)))
    out = _gather(table, tail, idx)
    return out.reshape(x.shape[0], x.shape[1], EMBED_DIM)


# speedup vs baseline: 1.4464x; 1.0010x over previous
import jax
import jax.numpy as jnp
from jax import lax
from jax.experimental import pallas as pl
from jax.experimental.pallas import tpu as pltpu
from jax.experimental.pallas import tpu_sc as plsc

NUM_WORDS = 100000
EMBED_DIM = 300

NC = 2
NS = 16
NW = NC * NS

TOT = 4096 * 50
PER_W = TOT // NW
CH = 128
NCH = PER_W // CH
TAIL = EMBED_DIM - 256


def _gather_body(table_hbm, tail_hbm, idx_hbm, out_hbm, idx_v, rows_v, b2, sem):
    wid = lax.axis_index("s") * NC + lax.axis_index("c")
    base = wid * PER_W
    lanes = lax.iota(jnp.int32, 16)
    tail_mask = lanes < (TAIL - 32)

    def chunk(i, carry):
        off = pl.multiple_of(base + i * CH, CH)
        pltpu.sync_copy(idx_hbm.at[pl.ds(off, CH)], idx_v)
        c0 = pltpu.async_copy(
            table_hbm.at[idx_v, pl.ds(0, 128)], rows_v.at[:, pl.ds(0, 128)], sem)
        c1 = pltpu.async_copy(
            table_hbm.at[idx_v, pl.ds(128, 128)], rows_v.at[:, pl.ds(128, 128)],
            sem)
        c2 = pltpu.async_copy(tail_hbm.at[idx_v], b2, sem)
        c0.wait()
        c1.wait()
        c2.wait()

        def row_fix(j, c):
            jv = jnp.full((16,), j, jnp.int32)
            v0 = b2[j, pl.ds(0, 16)]
            v1 = b2[j, pl.ds(16, 16)]
            v2 = b2[j, pl.ds(32, 16)]
            plsc.store_scatter(rows_v, [jv, lanes + 256], v0)
            plsc.store_scatter(rows_v, [jv, lanes + 272], v1)
            plsc.store_scatter(rows_v, [jv, lanes + 288], v2, mask=tail_mask)
            return c

        lax.fori_loop(0, CH, row_fix, 0)
        pltpu.sync_copy(rows_v, out_hbm.at[pl.ds(off, CH)])
        return carry

    lax.fori_loop(0, NCH, chunk, 0)


_gather = pl.kernel(
    _gather_body,
    out_type=jax.ShapeDtypeStruct((TOT, EMBED_DIM), jnp.float32),
    mesh=plsc.VectorSubcoreMesh(
        core_axis_name="c", subcore_axis_name="s", num_cores=NC, num_subcores=NS
    ),
    scratch_types=[
        pltpu.VMEM((CH,), jnp.int32),
        pltpu.VMEM((CH, EMBED_DIM), jnp.float32),
        pltpu.VMEM((CH, 128), jnp.float32),
        pltpu.SemaphoreType.DMA,
    ],
    compiler_params=pltpu.CompilerParams(needs_layout_passes=False),
)


@jax.jit
def kernel(x, table):
    idx = x.reshape(-1).astype(jnp.int32)
    one = lax.optimization_barrier(jnp.float32(1.0))
    tail = jnp.pad(lax.slice(table, (0, 256), (NUM_WORDS, EMBED_DIM)),
                   ((0, 0), (0, 128 - TAIL))) * one
    out = _gather(table, tail, idx)
    return out.reshape(x.shape[0], x.shape[1], EMBED_DIM)


# speedup vs baseline: 1.4868x; 1.0280x over previous
import jax
import jax.numpy as jnp
from jax import lax
from jax.experimental import pallas as pl
from jax.experimental.pallas import tpu as pltpu
from jax.experimental.pallas import tpu_sc as plsc

NUM_WORDS = 100000
EMBED_DIM = 300

NC = 2
NS = 16
NW = NC * NS

TOT = 4096 * 50
PER_W = TOT // NW
CH = 128
NCH = PER_W // CH
TAIL = EMBED_DIM - 256


def _gather_body(table_hbm, tail_hbm, idx_hbm, out_hbm, idx_v, rows_v, b2, sem):
    wid = lax.axis_index("s") * NC + lax.axis_index("c")
    base = wid * PER_W
    lanes = lax.iota(jnp.int32, 16)
    tail_mask = lanes < (TAIL - 32)

    def chunk(i, carry):
        off = pl.multiple_of(base + i * CH, CH)
        pltpu.sync_copy(idx_hbm.at[pl.ds(off, CH)], idx_v)
        c0 = pltpu.async_copy(
            table_hbm.at[idx_v, pl.ds(0, 128)], rows_v.at[:, pl.ds(0, 128)], sem)
        c1 = pltpu.async_copy(
            table_hbm.at[idx_v, pl.ds(128, 128)], rows_v.at[:, pl.ds(128, 128)],
            sem)
        c2 = pltpu.async_copy(tail_hbm.at[idx_v], b2, sem)
        c0.wait()
        c1.wait()
        c2.wait()

        def row_fix(j, c):
            jv = jnp.full((16,), j, jnp.int32)
            v0 = b2[j, pl.ds(0, 16)]
            v1 = b2[j, pl.ds(16, 16)]
            v2 = b2[j, pl.ds(32, 16)]
            plsc.store_scatter(rows_v, [jv, lanes + 256], v0)
            plsc.store_scatter(rows_v, [jv, lanes + 272], v1)
            plsc.store_scatter(rows_v, [jv, lanes + 288], v2, mask=tail_mask)
            return c

        lax.fori_loop(0, CH, row_fix, 0)
        pltpu.sync_copy(rows_v, out_hbm.at[pl.ds(off, CH)])
        return carry

    lax.fori_loop(0, NCH, chunk, 0)


_gather = pl.kernel(
    _gather_body,
    out_type=jax.ShapeDtypeStruct((TOT, EMBED_DIM), jnp.float32),
    mesh=plsc.VectorSubcoreMesh(
        core_axis_name="c", subcore_axis_name="s", num_cores=NC, num_subcores=NS
    ),
    scratch_types=[
        pltpu.VMEM((CH,), jnp.int32),
        pltpu.VMEM((CH, EMBED_DIM), jnp.float32),
        pltpu.VMEM((CH, 128), jnp.float32),
        pltpu.SemaphoreType.DMA,
    ],
    compiler_params=pltpu.CompilerParams(needs_layout_passes=False),
)


_TAIL_RB = 4000


def _tail_copy_body(t_ref, o_ref):
    o_ref[...] = t_ref[...]


_tail_copy = pl.pallas_call(
    _tail_copy_body,
    grid=(NUM_WORDS // _TAIL_RB,),
    in_specs=[pl.BlockSpec((_TAIL_RB, 128), lambda i: (i, 2))],
    out_specs=pl.BlockSpec((_TAIL_RB, 128), lambda i: (i, 0)),
    out_shape=jax.ShapeDtypeStruct((NUM_WORDS, 128), jnp.float32),
)


@jax.jit
def kernel(x, table):
    idx = x.reshape(-1).astype(jnp.int32)
    tail = _tail_copy(table)
    out = _gather(table, tail, idx)
    return out.reshape(x.shape[0], x.shape[1], EMBED_DIM)


# speedup vs baseline: 1.6472x; 1.1079x over previous
import jax
import jax.numpy as jnp
from jax import lax
from jax.experimental import pallas as pl
from jax.experimental.pallas import tpu as pltpu
from jax.experimental.pallas import tpu_sc as plsc

NUM_WORDS = 100000
EMBED_DIM = 300

NC = 2
NS = 16
NW = NC * NS

B = 4096
L = 50
LP = 56
PER_W = B // NW
TAIL = EMBED_DIM - 256


def _gather_body(table_hbm, tail_hbm, idx_hbm, out_hbm, idx_v, rows_v, b2, sem):
    wid = lax.axis_index("s") * NC + lax.axis_index("c")
    base = wid * PER_W
    lanes = lax.iota(jnp.int32, 16)
    tail_mask = lanes < (TAIL - 32)

    def chunk(k, carry):
        b = base + k
        pltpu.sync_copy(idx_hbm.at[pl.ds(pl.multiple_of(b * LP, 8), L)], idx_v)
        c0 = pltpu.async_copy(
            table_hbm.at[idx_v, pl.ds(0, 128)], rows_v.at[:, pl.ds(0, 128)], sem)
        c1 = pltpu.async_copy(
            table_hbm.at[idx_v, pl.ds(128, 128)], rows_v.at[:, pl.ds(128, 128)],
            sem)
        c2 = pltpu.async_copy(tail_hbm.at[idx_v], b2, sem)
        c0.wait()
        c1.wait()
        c2.wait()

        def row_fix(j, c):
            jv = jnp.full((16,), j, jnp.int32)
            v0 = b2[j, pl.ds(0, 16)]
            v1 = b2[j, pl.ds(16, 16)]
            v2 = b2[j, pl.ds(32, 16)]
            plsc.store_scatter(rows_v, [jv, lanes + 256], v0)
            plsc.store_scatter(rows_v, [jv, lanes + 272], v1)
            plsc.store_scatter(rows_v, [jv, lanes + 288], v2, mask=tail_mask)
            return c

        lax.fori_loop(0, L, row_fix, 0)
        pltpu.sync_copy(rows_v, out_hbm.at[b])
        return carry

    lax.fori_loop(0, PER_W, chunk, 0)


_gather = pl.kernel(
    _gather_body,
    out_type=jax.ShapeDtypeStruct((B, L, EMBED_DIM), jnp.float32),
    mesh=plsc.VectorSubcoreMesh(
        core_axis_name="c", subcore_axis_name="s", num_cores=NC, num_subcores=NS
    ),
    scratch_types=[
        pltpu.VMEM((L,), jnp.int32),
        pltpu.VMEM((L, EMBED_DIM), jnp.float32),
        pltpu.VMEM((L, 128), jnp.float32),
        pltpu.SemaphoreType.DMA,
    ],
    compiler_params=pltpu.CompilerParams(needs_layout_passes=False),
)

_TAIL_RB = 4000


def _tail_copy_body(t_ref, o_ref):
    o_ref[...] = t_ref[...]


_tail_copy = pl.pallas_call(
    _tail_copy_body,
    grid=(NUM_WORDS // _TAIL_RB,),
    in_specs=[pl.BlockSpec((_TAIL_RB, 128), lambda i: (i, 2))],
    out_specs=pl.BlockSpec((_TAIL_RB, 128), lambda i: (i, 0)),
    out_shape=jax.ShapeDtypeStruct((NUM_WORDS, 128), jnp.float32),
)


@jax.jit
def kernel(x, table):
    idx = jnp.pad(x.astype(jnp.int32), ((0, 0), (0, LP - L))).reshape(-1)
    tail = _tail_copy(table)
    out = _gather(table, tail, idx)
    return out


# speedup vs baseline: 2.0393x; 1.2381x over previous
import jax
import jax.numpy as jnp
from jax import lax
from jax.experimental import pallas as pl
from jax.experimental.pallas import tpu as pltpu
from jax.experimental.pallas import tpu_sc as plsc

NUM_WORDS = 100000
EMBED_DIM = 300

NC = 2
NS = 16
NW = NC * NS

B = 4096
L = 50
LP = 56
PER_W = B // NW
RING = 4
STEPS = PER_W // RING
TAIL = EMBED_DIM - 256


def _gather_body(table_hbm, tail_hbm, idx_hbm, out_hbm, idx_all,
                 rows0, rows1, rows2, rows3, b20, b21, b22, b23,
                 gs0, gs1, gs2, gs3, ws0, ws1, ws2, ws3):
    rows = (rows0, rows1, rows2, rows3)
    b2s = (b20, b21, b22, b23)
    gsems = (gs0, gs1, gs2, gs3)
    wsems = (ws0, ws1, ws2, ws3)

    wid = lax.axis_index("s") * NC + lax.axis_index("c")
    base = wid * PER_W
    lanes = lax.iota(jnp.int32, 16)
    tail_mask = lanes < (TAIL - 32)

    pltpu.sync_copy(idx_hbm.at[pl.ds(pl.multiple_of(base * LP, 8), PER_W * LP)],
                    idx_all)

    def row_fix(rows_v, b2):
        def fix(j, c):
            jv = jnp.full((16,), j, jnp.int32)
            v0 = b2[j, pl.ds(0, 16)]
            v1 = b2[j, pl.ds(16, 16)]
            v2 = b2[j, pl.ds(32, 16)]
            plsc.store_scatter(rows_v, [jv, lanes + 256], v0)
            plsc.store_scatter(rows_v, [jv, lanes + 272], v1)
            plsc.store_scatter(rows_v, [jv, lanes + 288], v2, mask=tail_mask)
            return c

        lax.fori_loop(0, L, fix, 0)

    def step(t, carry):
        handles = []
        for i in range(RING):
            @pl.when(t > 0)
            def _(i=i):
                pltpu.make_async_copy(rows[i], out_hbm.at[0], wsems[i]).wait()

            iv = idx_all.at[pl.ds((t * RING + i) * LP, L)]
            c0 = pltpu.async_copy(
                table_hbm.at[iv, pl.ds(0, 128)],
                rows[i].at[:, pl.ds(0, 128)], gsems[i])
            c1 = pltpu.async_copy(
                table_hbm.at[iv, pl.ds(128, 128)],
                rows[i].at[:, pl.ds(128, 128)], gsems[i])
            c2 = pltpu.async_copy(tail_hbm.at[iv], b2s[i], gsems[i])
            handles.append((c0, c1, c2))
        for i in range(RING):
            c0, c1, c2 = handles[i]
            c0.wait()
            c1.wait()
            c2.wait()
            row_fix(rows[i], b2s[i])
            pltpu.async_copy(rows[i], out_hbm.at[base + t * RING + i], wsems[i])
        return carry

    lax.fori_loop(0, STEPS, step, 0)
    for i in range(RING):
        pltpu.make_async_copy(rows[i], out_hbm.at[0], wsems[i]).wait()


_gather = pl.kernel(
    _gather_body,
    out_type=jax.ShapeDtypeStruct((B, L, EMBED_DIM), jnp.float32),
    mesh=plsc.VectorSubcoreMesh(
        core_axis_name="c", subcore_axis_name="s", num_cores=NC, num_subcores=NS
    ),
    scratch_types=(
        [pltpu.VMEM((PER_W * LP,), jnp.int32)]
        + [pltpu.VMEM((L, EMBED_DIM), jnp.float32)] * RING
        + [pltpu.VMEM((L, 128), jnp.float32)] * RING
        + [pltpu.SemaphoreType.DMA] * (2 * RING)
    ),
    compiler_params=pltpu.CompilerParams(needs_layout_passes=False),
)

_TAIL_RB = 4000


def _tail_copy_body(t_ref, o_ref):
    o_ref[...] = t_ref[...]


_tail_copy = pl.pallas_call(
    _tail_copy_body,
    grid=(NUM_WORDS // _TAIL_RB,),
    in_specs=[pl.BlockSpec((_TAIL_RB, 128), lambda i: (i, 2))],
    out_specs=pl.BlockSpec((_TAIL_RB, 128), lambda i: (i, 0)),
    out_shape=jax.ShapeDtypeStruct((NUM_WORDS, 128), jnp.float32),
)


@jax.jit
def kernel(x, table):
    idx = jnp.pad(x.astype(jnp.int32), ((0, 0), (0, LP - L))).reshape(-1)
    tail = _tail_copy(table)
    return _gather(table, tail, idx)
